# baseline (speedup 1.0000x reference)
import functools

import jax
import jax.numpy as jnp
from jax import lax
from jax.experimental import pallas as pl
from jax.experimental.pallas import tpu as pltpu
from jax.experimental.pallas import tpu_sc as plsc

N = 10000
E = 320000
R = 8
D = 128

NC = 2
NS = 16
CB = 128
E2 = 327680
EPT = E2 // (NC * NS)
NCH = EPT // CB
CNT_PAD = 80128
_CNT_SLICE = CNT_PAD // NS

PN = 10240

_MASK14 = 16383
_PADVAL = 10000

_mesh = plsc.VectorSubcoreMesh(
    core_axis_name="c", subcore_axis_name="s", num_cores=NC, num_subcores=NS)


def _unpack_batch(p16):
    r16 = lax.shift_right_logical(p16, 28)
    s16 = lax.shift_right_logical(p16, 14) & _MASK14
    d16 = p16 & _MASK14
    gid = r16 * N + s16
    cid = lax.shift_left(d16, 3) | r16
    return gid, cid, d16


@functools.partial(
    pl.kernel,
    out_type=[
        jax.ShapeDtypeStruct((CNT_PAD,), jnp.float32),
        jax.ShapeDtypeStruct((CNT_PAD,), jnp.float32),
    ],
    mesh=_mesh,
    scratch_types=[
        pltpu.VMEM_SHARED((CNT_PAD,), jnp.float32),
        pltpu.VMEM((EPT,), jnp.int32),
        pltpu.VMEM((CB,), jnp.int32),
        pltpu.VMEM((CB,), jnp.int32),
        pltpu.VMEM((CB,), jnp.float32),
        pltpu.VMEM((_CNT_SLICE,), jnp.float32),
        pltpu.SemaphoreType.DMA,
        pltpu.SemaphoreType.DMA,
    ],
)
def _count_kernel(pk_hbm, cnt0_hbm, cnt1_hbm, counts_sh,
                  pk_v, ca_v, cb_v, ones_v, zb_v, sem_a, sem_b):
    c = lax.axis_index("c")
    s = lax.axis_index("s")
    wid = c * NS + s

    def zero_step(i, _):
        zb_v[pl.ds(i * 16, 16)] = jnp.zeros((16,), jnp.float32)
        return 0
    lax.fori_loop(0, _CNT_SLICE // 16, zero_step, 0)
    for k in range(CB // 16):
        ones_v[pl.ds(k * 16, 16)] = jnp.ones((16,), jnp.float32)

    pltpu.sync_copy(zb_v, counts_sh.at[pl.ds(s * _CNT_SLICE, _CNT_SLICE)])
    plsc.subcore_barrier()

    pltpu.sync_copy(pk_hbm.at[pl.ds(wid * EPT, EPT)], pk_v)

    bank = ((ca_v, sem_a), (cb_v, sem_b))

    def unpack_cid(i, buf):
        def ub(b, _):
            p16 = pk_v[pl.ds(i * CB + b * 16, 16)]
            _, cid, _ = _unpack_batch(p16)
            buf[pl.ds(b * 16, 16)] = cid
            return 0
        lax.fori_loop(0, CB // 16, ub, 0)

    def pair(k, _):
        for j in range(2):
            buf, sem = bank[j]

            @pl.when(k > 0)
            def _():
                pltpu.make_async_copy(ones_v, counts_sh.at[buf], sem).wait()
            unpack_cid(2 * k + j, buf)
            pltpu.async_copy(ones_v, counts_sh.at[buf], sem, add=True)
        return 0
    lax.fori_loop(0, NCH // 2, pair, 0)
    for j in range(2):
        buf, sem = bank[j]
        pltpu.make_async_copy(ones_v, counts_sh.at[buf], sem).wait()

    plsc.subcore_barrier()
    out = [cnt0_hbm, cnt1_hbm]
    for cc in range(NC):
        @pl.when(c == cc)
        def _(cc=cc):
            pltpu.sync_copy(counts_sh.at[pl.ds(s * _CNT_SLICE, _CNT_SLICE)],
                            zb_v)
            pltpu.sync_copy(zb_v,
                            out[cc].at[pl.ds(s * _CNT_SLICE, _CNT_SLICE)])


@functools.partial(
    pl.kernel,
    out_type=jax.ShapeDtypeStruct((NC, PN, D), jnp.float32),
    mesh=_mesh,
    scratch_types=[
        pltpu.VMEM_SHARED((PN, D), jnp.float32),
        pltpu.VMEM((EPT,), jnp.int32),
        pltpu.VMEM((CB, D), jnp.float32),
        pltpu.VMEM((CB, D), jnp.float32),
        pltpu.VMEM((CB,), jnp.int32),
        pltpu.VMEM((CB,), jnp.int32),
        pltpu.VMEM((CB,), jnp.int32),
        pltpu.VMEM((CB,), jnp.int32),
        pltpu.VMEM((CB,), jnp.int32),
        pltpu.VMEM((CB,), jnp.int32),
        pltpu.VMEM((CB,), jnp.float32),
        pltpu.VMEM((CB,), jnp.float32),
        pltpu.SemaphoreType.DMA,
        pltpu.SemaphoreType.DMA,
        pltpu.SemaphoreType.DMA,
        pltpu.SemaphoreType.DMA,
    ],
)
def _msg_kernel(nrm_hbm, a_hbm, pk_hbm, parts_hbm,
                agg_sh, pk_v, rows_a, rows_b, ga_v, gb_v, na_v, nb_v,
                da_v, db_v, fa_v, fb_v, sga, sgb, ssa, ssb):
    c = lax.axis_index("c")
    s = lax.axis_index("s")
    wid = c * NS + s

    zrows = PN // NS // 5

    def zero_step(i, _):
        for j in range(D // 16):
            rows_a[i, pl.ds(j * 16, 16)] = jnp.zeros((16,), jnp.float32)
        return 0
    lax.fori_loop(0, CB, zero_step, 0)
    for k in range(5):
        pltpu.sync_copy(
            rows_a, agg_sh.at[pl.ds(s * (PN // NS) + k * zrows, zrows)])
    plsc.subcore_barrier()

    pltpu.sync_copy(pk_hbm.at[pl.ds(wid * EPT, EPT)], pk_v)

    bank = (
        (rows_a, ga_v, na_v, da_v, fa_v, sga, ssa),
        (rows_b, gb_v, nb_v, db_v, fb_v, sgb, ssb),
    )

    def unpack(i, b):
        _, gbuf, nbuf, dbuf, _, _, _ = bank[b]

        def ub(bi, _):
            p16 = pk_v[pl.ds(i * CB + bi * 16, 16)]
            gid, cid, d16 = _unpack_batch(p16)
            gbuf[pl.ds(bi * 16, 16)] = gid
            nbuf[pl.ds(bi * 16, 16)] = cid
            dbuf[pl.ds(bi * 16, 16)] = d16
            return 0
        lax.fori_loop(0, CB // 16, ub, 0)

    def issue_gath(b):
        rows_v, gbuf, nbuf, _, fbuf, sg, _ = bank[b]
        pltpu.async_copy(a_hbm.at[gbuf], rows_v, sg)
        pltpu.async_copy(nrm_hbm.at[nbuf], fbuf, sg)

    def wait_gath(b):
        rows_v, _, _, _, fbuf, sg, _ = bank[b]
        pltpu.make_async_copy(a_hbm.at[pl.ds(0, CB)], rows_v, sg).wait()
        pltpu.make_async_copy(nrm_hbm.at[pl.ds(0, CB)], fbuf, sg).wait()

    def wait_scat(b):
        rows_v, _, _, dbuf, _, _, ss = bank[b]
        pltpu.make_async_copy(rows_v, agg_sh.at[dbuf], ss).wait()

    def scale_and_scatter(b):
        rows_v, _, _, dbuf, fbuf, _, ss = bank[b]

        def scale_batch(bi, _):
            n16 = fbuf[pl.ds(bi * 16, 16)]
            for el in range(16):
                nb = jnp.broadcast_to(lax.slice_in_dim(n16, el, el + 1), (16,))
                row = bi * 16 + el
                for j in range(D // 16):
                    rows_v[row, pl.ds(j * 16, 16)] = (
                        rows_v[row, pl.ds(j * 16, 16)] * nb)
            return 0
        lax.fori_loop(0, CB // 16, scale_batch, 0)
        pltpu.async_copy(rows_v, agg_sh.at[dbuf], ss, add=True)

    unpack(0, 0)
    issue_gath(0)

    def pair(k, _):
        @pl.when(k > 0)
        def _():
            wait_scat(1)
        unpack(2 * k + 1, 1)
        issue_gath(1)
        wait_gath(0)
        scale_and_scatter(0)
        wait_gath(1)

        @pl.when(k < (NCH // 2) - 1)
        def _():
            wait_scat(0)
            unpack(2 * k + 2, 0)
            issue_gath(0)
        scale_and_scatter(1)
        return 0
    lax.fori_loop(0, NCH // 2, pair, 0)
    wait_scat(0)
    wait_scat(1)

    plsc.subcore_barrier()
    for k in range(5):
        sl = pl.ds(s * (PN // NS) + k * zrows, zrows)
        pltpu.sync_copy(agg_sh.at[sl], rows_a)
        pltpu.sync_copy(rows_a, parts_hbm.at[c].at[sl])


_BN = 1000
_PB = 8


def _nrm_body(a_ref, b_ref, o_ref):
    o_ref[...] = 1.0 / jnp.maximum(a_ref[...] + b_ref[...], 1.0)


def _nrm_call(c0, c1):
    r2 = (CNT_PAD // D, D)
    out = pl.pallas_call(
        _nrm_body,
        out_shape=jax.ShapeDtypeStruct(r2, jnp.float32),
    )(c0.reshape(r2), c1.reshape(r2))
    return out.reshape(CNT_PAD)


_PW = 320


def _pad_body(p_ref, o_ref):
    nin = E // _PW // _PB

    @pl.when(pl.program_id(0) < nin)
    def _():
        o_ref[...] = p_ref[...]

    @pl.when(pl.program_id(0) >= nin)
    def _():
        pos = (lax.broadcasted_iota(jnp.int32, (_PB, _PW), 0) * _PW
               + lax.broadcasted_iota(jnp.int32, (_PB, _PW), 1))
        o_ref[...] = _PADVAL + (pos & 127)


def _pad_call(packed):
    nin = E // _PW // _PB
    nout = E2 // _PW // _PB
    out = pl.pallas_call(
        _pad_body,
        grid=(nout,),
        in_specs=[
            pl.BlockSpec((_PB, _PW), lambda i: (jnp.minimum(i, nin - 1), 0))],
        out_specs=pl.BlockSpec((_PB, _PW), lambda i: (i, 0)),
        out_shape=jax.ShapeDtypeStruct((E2 // _PW, _PW), jnp.int32),
    )(packed.reshape(E // _PW, _PW))
    return out.reshape(E2)


def _mm1_body(x_ref, w_ref, ws_ref, o_ref, os_ref):
    x = x_ref[...]
    o_ref[0] = jnp.dot(x, w_ref[0], preferred_element_type=jnp.float32)

    @pl.when(pl.program_id(1) == 0)
    def _():
        os_ref[...] = jnp.dot(x, ws_ref[...],
                              preferred_element_type=jnp.float32)


def _mm1_call(h, w, wself):
    return pl.pallas_call(
        _mm1_body,
        grid=(N // _BN, R),
        in_specs=[
            pl.BlockSpec((_BN, D), lambda i, r: (i, 0)),
            pl.BlockSpec((1, D, D), lambda i, r: (r, 0, 0)),
            pl.BlockSpec((D, D), lambda i, r: (0, 0)),
        ],
        out_specs=[
            pl.BlockSpec((1, _BN, D), lambda i, r: (r, i, 0)),
            pl.BlockSpec((_BN, D), lambda i, r: (i, 0)),
        ],
        out_shape=[
            jax.ShapeDtypeStruct((R, N, D), jnp.float32),
            jax.ShapeDtypeStruct((N, D), jnp.float32),
        ],
    )(h, w, wself)


def _mm2_body(p_ref, s_ref, b_ref, w_ref, ws_ref, o_ref, os_ref):
    h = p_ref[0] + p_ref[1] + s_ref[...] + b_ref[0]
    o_ref[0] = jnp.dot(h, w_ref[0], preferred_element_type=jnp.float32)

    @pl.when(pl.program_id(1) == 0)
    def _():
        os_ref[...] = jnp.dot(h, ws_ref[...],
                              preferred_element_type=jnp.float32)


def _mm2_call(parts, aself, b_prev, w, wself):
    return pl.pallas_call(
        _mm2_body,
        grid=(N // _BN, R),
        in_specs=[
            pl.BlockSpec((NC, _BN, D), lambda i, r: (0, i, 0)),
            pl.BlockSpec((_BN, D), lambda i, r: (i, 0)),
            pl.BlockSpec((1, D), lambda i, r: (0, 0)),
            pl.BlockSpec((1, D, D), lambda i, r: (r, 0, 0)),
            pl.BlockSpec((D, D), lambda i, r: (0, 0)),
        ],
        out_specs=[
            pl.BlockSpec((1, _BN, D), lambda i, r: (r, i, 0)),
            pl.BlockSpec((_BN, D), lambda i, r: (i, 0)),
        ],
        out_shape=[
            jax.ShapeDtypeStruct((R, N, D), jnp.float32),
            jax.ShapeDtypeStruct((N, D), jnp.float32),
        ],
    )(parts, aself, b_prev, w, wself)


def _fin_body(p_ref, s_ref, b_ref, o_ref):
    o_ref[...] = p_ref[0] + p_ref[1] + s_ref[...] + b_ref[0]


def _fin_call(parts, aself, b_prev):
    return pl.pallas_call(
        _fin_body,
        grid=(N // _BN,),
        in_specs=[
            pl.BlockSpec((NC, _BN, D), lambda i: (0, i, 0)),
            pl.BlockSpec((_BN, D), lambda i: (i, 0)),
            pl.BlockSpec((1, D), lambda i: (0, 0)),
        ],
        out_specs=pl.BlockSpec((_BN, D), lambda i: (i, 0)),
        out_shape=jax.ShapeDtypeStruct((N, D), jnp.float32),
    )(parts, aself, b_prev)


@jax.jit
def kernel(node_feats, edge_index, rel_ids, W1, Wself1, b1, W2, Wself2, b2):
    src = edge_index[0].astype(jnp.int32)
    dst = edge_index[1].astype(jnp.int32)
    rid = rel_ids.astype(jnp.int32)

    packed = _pad_call(
        lax.shift_left(rid, 28) | lax.shift_left(src, 14) | dst)

    cnt0, cnt1 = _count_kernel(packed)
    nrm = _nrm_call(cnt0, cnt1)

    b1r = b1.reshape(1, D)
    b2r = b2.reshape(1, D)

    a1, s1 = _mm1_call(node_feats, W1, Wself1)
    parts1 = _msg_kernel(nrm, a1.reshape(R * N, D), packed)
    a2, s2 = _mm2_call(parts1, s1, b1r, W2, Wself2)
    parts2 = _msg_kernel(nrm, a2.reshape(R * N, D), packed)
    return _fin_call(parts2, s2, b2r)


# speedup vs baseline: 2.2545x; 2.2545x over previous
import functools

import jax
import jax.numpy as jnp
from jax import lax
from jax.experimental import pallas as pl
from jax.experimental.pallas import tpu as pltpu
from jax.experimental.pallas import tpu_sc as plsc

N = 10000
E = 320000
R = 8
D = 128

NC = 2
NS = 16
CB = 128
E2 = 327680
EPT = E2 // (NC * NS)
NCH = EPT // CB
CNT_PAD = 81152
_CNT_SLICE = CNT_PAD // NS

PN = 10240

_MASK14 = 16383
_PADVAL = 10000

_mesh = plsc.VectorSubcoreMesh(
    core_axis_name="c", subcore_axis_name="s", num_cores=NC, num_subcores=NS)


def _unpack_batch(p16):
    r16 = lax.shift_right_logical(p16, 28)
    s16 = lax.shift_right_logical(p16, 14) & _MASK14
    d16 = p16 & _MASK14
    gid = r16 * N + s16
    cid = lax.shift_left(d16, 3) | r16
    return gid, cid, d16


@functools.partial(
    pl.kernel,
    out_type=[
        jax.ShapeDtypeStruct((CNT_PAD,), jnp.float32),
        jax.ShapeDtypeStruct((CNT_PAD,), jnp.float32),
    ],
    mesh=_mesh,
    scratch_types=[
        pltpu.VMEM_SHARED((CNT_PAD,), jnp.float32),
        pltpu.VMEM((EPT,), jnp.int32),
        pltpu.VMEM((CB,), jnp.int32),
        pltpu.VMEM((CB,), jnp.int32),
        pltpu.VMEM((CB,), jnp.float32),
        pltpu.VMEM((_CNT_SLICE,), jnp.float32),
        pltpu.SemaphoreType.DMA,
        pltpu.SemaphoreType.DMA,
    ],
)
def _count_kernel(pk_hbm, cnt0_hbm, cnt1_hbm, counts_sh,
                  pk_v, ca_v, cb_v, ones_v, zb_v, sem_a, sem_b):
    c = lax.axis_index("c")
    s = lax.axis_index("s")
    wid = c * NS + s

    def zero_step(i, _):
        zb_v[pl.ds(i * 16, 16)] = jnp.zeros((16,), jnp.float32)
        return 0
    lax.fori_loop(0, _CNT_SLICE // 16, zero_step, 0)
    for k in range(CB // 16):
        ones_v[pl.ds(k * 16, 16)] = jnp.ones((16,), jnp.float32)

    pltpu.sync_copy(zb_v, counts_sh.at[pl.ds(s * _CNT_SLICE, _CNT_SLICE)])
    plsc.subcore_barrier()

    pltpu.sync_copy(pk_hbm.at[pl.ds(wid * EPT, EPT)], pk_v)

    bank = ((ca_v, sem_a), (cb_v, sem_b))

    def unpack_cid(i, buf):
        def ub(b, _):
            p16 = pk_v[pl.ds(i * CB + b * 16, 16)]
            _, cid, _ = _unpack_batch(p16)
            buf[pl.ds(b * 16, 16)] = cid
            return 0
        lax.fori_loop(0, CB // 16, ub, 0)

    def pair(k, _):
        for j in range(2):
            buf, sem = bank[j]

            @pl.when(k > 0)
            def _():
                pltpu.make_async_copy(ones_v, counts_sh.at[buf], sem).wait()
            unpack_cid(2 * k + j, buf)
            pltpu.async_copy(ones_v, counts_sh.at[buf], sem, add=True)
        return 0
    lax.fori_loop(0, NCH // 2, pair, 0)
    for j in range(2):
        buf, sem = bank[j]
        pltpu.make_async_copy(ones_v, counts_sh.at[buf], sem).wait()

    plsc.subcore_barrier()
    out = [cnt0_hbm, cnt1_hbm]
    for cc in range(NC):
        @pl.when(c == cc)
        def _(cc=cc):
            pltpu.sync_copy(counts_sh.at[pl.ds(s * _CNT_SLICE, _CNT_SLICE)],
                            zb_v)
            pltpu.sync_copy(zb_v,
                            out[cc].at[pl.ds(s * _CNT_SLICE, _CNT_SLICE)])


@functools.partial(
    pl.kernel,
    out_type=jax.ShapeDtypeStruct((NC, PN, D), jnp.float32),
    mesh=_mesh,
    scratch_types=[
        pltpu.VMEM_SHARED((PN, D), jnp.float32),
        pltpu.VMEM((EPT,), jnp.int32),
        pltpu.VMEM((CB, D), jnp.float32),
        pltpu.VMEM((CB, D), jnp.float32),
        pltpu.VMEM((CB,), jnp.int32),
        pltpu.VMEM((CB,), jnp.int32),
        pltpu.VMEM((CB,), jnp.int32),
        pltpu.VMEM((CB,), jnp.int32),
        pltpu.VMEM((CB,), jnp.int32),
        pltpu.VMEM((CB,), jnp.int32),
        pltpu.VMEM((CB,), jnp.float32),
        pltpu.VMEM((CB,), jnp.float32),
        pltpu.SemaphoreType.DMA,
        pltpu.SemaphoreType.DMA,
        pltpu.SemaphoreType.DMA,
        pltpu.SemaphoreType.DMA,
    ],
)
def _msg_kernel(nrm_hbm, a_hbm, pk_hbm, parts_hbm,
                agg_sh, pk_v, rows_a, rows_b, ga_v, gb_v, na_v, nb_v,
                da_v, db_v, fa_v, fb_v, sga, sgb, ssa, ssb):
    c = lax.axis_index("c")
    s = lax.axis_index("s")
    wid = c * NS + s

    zrows = PN // NS // 5

    def zero_step(i, _):
        for j in range(D // 16):
            rows_a[i, pl.ds(j * 16, 16)] = jnp.zeros((16,), jnp.float32)
        return 0
    lax.fori_loop(0, CB, zero_step, 0)
    for k in range(5):
        pltpu.sync_copy(
            rows_a, agg_sh.at[pl.ds(s * (PN // NS) + k * zrows, zrows)])
    plsc.subcore_barrier()

    pltpu.sync_copy(pk_hbm.at[pl.ds(wid * EPT, EPT)], pk_v)

    bank = (
        (rows_a, ga_v, na_v, da_v, fa_v, sga, ssa),
        (rows_b, gb_v, nb_v, db_v, fb_v, sgb, ssb),
    )

    def unpack(i, b):
        _, gbuf, nbuf, dbuf, _, _, _ = bank[b]

        def ub(bi, _):
            p16 = pk_v[pl.ds(i * CB + bi * 16, 16)]
            gid, cid, d16 = _unpack_batch(p16)
            gbuf[pl.ds(bi * 16, 16)] = gid
            nbuf[pl.ds(bi * 16, 16)] = cid
            dbuf[pl.ds(bi * 16, 16)] = d16
            return 0
        lax.fori_loop(0, CB // 16, ub, 0)

    def issue_gath(b):
        rows_v, gbuf, nbuf, _, fbuf, sg, _ = bank[b]
        pltpu.async_copy(a_hbm.at[gbuf], rows_v, sg)
        pltpu.async_copy(nrm_hbm.at[nbuf], fbuf, sg)

    def wait_gath(b):
        rows_v, _, _, _, fbuf, sg, _ = bank[b]
        pltpu.make_async_copy(a_hbm.at[pl.ds(0, CB)], rows_v, sg).wait()
        pltpu.make_async_copy(nrm_hbm.at[pl.ds(0, CB)], fbuf, sg).wait()

    def wait_scat(b):
        rows_v, _, _, dbuf, _, _, ss = bank[b]
        pltpu.make_async_copy(rows_v, agg_sh.at[dbuf], ss).wait()

    def scale_and_scatter(b):
        rows_v, _, _, dbuf, fbuf, _, ss = bank[b]

        def scale_batch(bi, _):
            n16 = fbuf[pl.ds(bi * 16, 16)]
            for el in range(16):
                nb = jnp.broadcast_to(lax.slice_in_dim(n16, el, el + 1), (16,))
                row = bi * 16 + el
                for j in range(D // 16):
                    rows_v[row, pl.ds(j * 16, 16)] = (
                        rows_v[row, pl.ds(j * 16, 16)] * nb)
            return 0
        lax.fori_loop(0, CB // 16, scale_batch, 0)
        pltpu.async_copy(rows_v, agg_sh.at[dbuf], ss, add=True)

    unpack(0, 0)
    issue_gath(0)

    def pair(k, _):
        @pl.when(k > 0)
        def _():
            wait_scat(1)
        unpack(2 * k + 1, 1)
        issue_gath(1)
        wait_gath(0)
        scale_and_scatter(0)
        wait_gath(1)

        @pl.when(k < (NCH // 2) - 1)
        def _():
            wait_scat(0)
            unpack(2 * k + 2, 0)
            issue_gath(0)
        scale_and_scatter(1)
        return 0
    lax.fori_loop(0, NCH // 2, pair, 0)
    wait_scat(0)
    wait_scat(1)

    plsc.subcore_barrier()
    for k in range(5):
        sl = pl.ds(s * (PN // NS) + k * zrows, zrows)
        pltpu.sync_copy(agg_sh.at[sl], rows_a)
        pltpu.sync_copy(rows_a, parts_hbm.at[c].at[sl])


_BN = 1000
_PB = 8


def _nrm_body(a_ref, b_ref, o_ref):
    o_ref[...] = 1.0 / jnp.maximum(a_ref[...] + b_ref[...], 1.0)


def _nrm_call(c0, c1):
    r2 = (CNT_PAD // D, D)
    out = pl.pallas_call(
        _nrm_body,
        out_shape=jax.ShapeDtypeStruct(r2, jnp.float32),
    )(c0.reshape(r2), c1.reshape(r2))
    return out.reshape(CNT_PAD)


_PW = 320


def _pad_body(p_ref, o_ref):
    nin = E // _PW // _PB

    @pl.when(pl.program_id(0) < nin)
    def _():
        o_ref[...] = p_ref[...]

    @pl.when(pl.program_id(0) >= nin)
    def _():
        pos = (lax.broadcasted_iota(jnp.int32, (_PB, _PW), 0) * _PW
               + lax.broadcasted_iota(jnp.int32, (_PB, _PW), 1)) & 127
        o_ref[...] = lax.shift_left(pos, 14) | (_PADVAL + pos)


def _pad_call(packed):
    nin = E // _PW // _PB
    nout = E2 // _PW // _PB
    out = pl.pallas_call(
        _pad_body,
        grid=(nout,),
        in_specs=[
            pl.BlockSpec((_PB, _PW), lambda i: (jnp.minimum(i, nin - 1), 0))],
        out_specs=pl.BlockSpec((_PB, _PW), lambda i: (i, 0)),
        out_shape=jax.ShapeDtypeStruct((E2 // _PW, _PW), jnp.int32),
    )(packed.reshape(E // _PW, _PW))
    return out.reshape(E2)


def _mm1_body(x_ref, w_ref, ws_ref, o_ref, os_ref):
    x = x_ref[...]
    o_ref[0] = jnp.dot(x, w_ref[0], preferred_element_type=jnp.float32)

    @pl.when(pl.program_id(1) == 0)
    def _():
        os_ref[...] = jnp.dot(x, ws_ref[...],
                              preferred_element_type=jnp.float32)


def _mm1_call(h, w, wself):
    return pl.pallas_call(
        _mm1_body,
        grid=(N // _BN, R),
        in_specs=[
            pl.BlockSpec((_BN, D), lambda i, r: (i, 0)),
            pl.BlockSpec((1, D, D), lambda i, r: (r, 0, 0)),
            pl.BlockSpec((D, D), lambda i, r: (0, 0)),
        ],
        out_specs=[
            pl.BlockSpec((1, _BN, D), lambda i, r: (r, i, 0)),
            pl.BlockSpec((_BN, D), lambda i, r: (i, 0)),
        ],
        out_shape=[
            jax.ShapeDtypeStruct((R, N, D), jnp.float32),
            jax.ShapeDtypeStruct((N, D), jnp.float32),
        ],
    )(h, w, wself)


def _mm2_body(p_ref, s_ref, b_ref, w_ref, ws_ref, o_ref, os_ref):
    h = p_ref[0] + p_ref[1] + s_ref[...] + b_ref[0]
    o_ref[0] = jnp.dot(h, w_ref[0], preferred_element_type=jnp.float32)

    @pl.when(pl.program_id(1) == 0)
    def _():
        os_ref[...] = jnp.dot(h, ws_ref[...],
                              preferred_element_type=jnp.float32)


def _mm2_call(parts, aself, b_prev, w, wself):
    return pl.pallas_call(
        _mm2_body,
        grid=(N // _BN, R),
        in_specs=[
            pl.BlockSpec((NC, _BN, D), lambda i, r: (0, i, 0)),
            pl.BlockSpec((_BN, D), lambda i, r: (i, 0)),
            pl.BlockSpec((1, D), lambda i, r: (0, 0)),
            pl.BlockSpec((1, D, D), lambda i, r: (r, 0, 0)),
            pl.BlockSpec((D, D), lambda i, r: (0, 0)),
        ],
        out_specs=[
            pl.BlockSpec((1, _BN, D), lambda i, r: (r, i, 0)),
            pl.BlockSpec((_BN, D), lambda i, r: (i, 0)),
        ],
        out_shape=[
            jax.ShapeDtypeStruct((R, N, D), jnp.float32),
            jax.ShapeDtypeStruct((N, D), jnp.float32),
        ],
    )(parts, aself, b_prev, w, wself)


def _fin_body(p_ref, s_ref, b_ref, o_ref):
    o_ref[...] = p_ref[0] + p_ref[1] + s_ref[...] + b_ref[0]


def _fin_call(parts, aself, b_prev):
    return pl.pallas_call(
        _fin_body,
        grid=(N // _BN,),
        in_specs=[
            pl.BlockSpec((NC, _BN, D), lambda i: (0, i, 0)),
            pl.BlockSpec((_BN, D), lambda i: (i, 0)),
            pl.BlockSpec((1, D), lambda i: (0, 0)),
        ],
        out_specs=pl.BlockSpec((_BN, D), lambda i: (i, 0)),
        out_shape=jax.ShapeDtypeStruct((N, D), jnp.float32),
    )(parts, aself, b_prev)


@jax.jit
def kernel(node_feats, edge_index, rel_ids, W1, Wself1, b1, W2, Wself2, b2):
    src = edge_index[0].astype(jnp.int32)
    dst = edge_index[1].astype(jnp.int32)
    rid = rel_ids.astype(jnp.int32)

    packed = _pad_call(
        lax.shift_left(rid, 28) | lax.shift_left(src, 14) | dst)

    cnt0, cnt1 = _count_kernel(packed)
    nrm = _nrm_call(cnt0, cnt1)

    b1r = b1.reshape(1, D)
    b2r = b2.reshape(1, D)

    a1, s1 = _mm1_call(node_feats, W1, Wself1)
    parts1 = _msg_kernel(nrm, a1.reshape(R * N, D), packed)
    a2, s2 = _mm2_call(parts1, s1, b1r, W2, Wself2)
    parts2 = _msg_kernel(nrm, a2.reshape(R * N, D), packed)
    return _fin_call(parts2, s2, b2r)
